# baseline (speedup 1.0000x reference)
import functools

import jax
import jax.numpy as jnp
from jax import lax
from jax.experimental import pallas as pl
from jax.experimental.pallas import tpu as pltpu
from jax.experimental.pallas import tpu_sc as plsc

N_ROWS = 16384
N_CLS = 2048
EMB = 1024

NUM_SC = 2
NUM_TILES = 16
NUM_WORKERS = NUM_SC * NUM_TILES
LANES = 16

COL_CH = 128
N_CGRP = EMB // COL_CH
N_QGRP = NUM_WORKERS // N_CGRP
CLS_PER_Q = N_CLS // N_QGRP
K_CH = 64
N_CH = N_ROWS // K_CH


def _sc_body(x_hbm, y2_hbm, seg_hbm,
             y_v, buf0, buf1, buf2, buf3, acc_v,
             sem0, sem1, sem2, sem3, ysem0, ysem1, ysem2, ysem3):
    c = lax.axis_index("c")
    s = lax.axis_index("s")
    w = c * NUM_TILES + s
    q = w % N_QGRP
    col0 = pl.multiple_of((w // N_QGRP) * COL_CH, COL_CH)
    cls0 = q * CLS_PER_Q
    zero16 = jnp.zeros((LANES,), jnp.float32)
    lane = lax.iota(jnp.int32, LANES)
    cols = [lane + j * LANES for j in range(COL_CH // LANES)]

    def _zacc(i, carry):
        for j in range(COL_CH // LANES):
            acc_v[i, pl.ds(j * LANES, LANES)] = zero16
        return carry
    lax.fori_loop(0, CLS_PER_Q + 8, _zacc, 0)

    NB = 4
    bufs = (buf0, buf1, buf2, buf3)
    sems = (sem0, sem1, sem2, sem3)
    ysems = (ysem0, ysem1, ysem2, ysem3)

    def _copy(t, p):
        return pltpu.make_async_copy(
            x_hbm.at[pl.ds(t * K_CH, K_CH), pl.ds(col0, COL_CH)],
            bufs[p], sems[p])

    def _ycopy(t, p):
        return pltpu.make_async_copy(y2_hbm.at[t], y_v.at[p], ysems[p])

    def _consume(t, p):
        buf = bufs[p]

        @plsc.parallel_loop(0, K_CH // LANES, unroll=2)
        def _grp(g16):
            yvec = y_v[p, pl.ds(g16 * LANES, LANES)]
            keep = (yvec >= cls0) & (yvec < cls0 + CLS_PER_Q)
            ylvec = jnp.where(keep, yvec - cls0, CLS_PER_Q)
            for l in range(LANES):
                yl = ylvec[l]
                ylv = jnp.full((LANES,), yl, jnp.int32)
                r = g16 * LANES + l
                for j in range(COL_CH // LANES):
                    plsc.addupdate_scatter(
                        acc_v, [ylv, cols[j]],
                        buf[r, pl.ds(j * LANES, LANES)])

    for t0 in range(NB):
        _copy(t0, t0).start()
        _ycopy(t0, t0).start()

    def _step(i, carry):
        for p in range(NB):
            t = NB * i + p
            _copy(t, p).wait()
            _ycopy(t, p).wait()
            _consume(t, p)

            @pl.when(t + NB < N_CH)
            def _(t=t, p=p):
                _copy(t + NB, p).start()
                _ycopy(t + NB, p).start()
        return carry
    lax.fori_loop(0, N_CH // NB, _step, 0)

    pltpu.sync_copy(acc_v.at[pl.ds(0, CLS_PER_Q)],
                    seg_hbm.at[pl.ds(cls0, CLS_PER_Q), pl.ds(col0, COL_CH)])


_sc_segsum = functools.partial(
    pl.kernel,
    out_type=jax.ShapeDtypeStruct((N_CLS, EMB), jnp.float32),
    mesh=plsc.VectorSubcoreMesh(core_axis_name="c", subcore_axis_name="s",
                                num_cores=NUM_SC, num_subcores=NUM_TILES),
    compiler_params=pltpu.CompilerParams(needs_layout_passes=False),
    scratch_types=[
        pltpu.VMEM((4, K_CH), jnp.int32),
        pltpu.VMEM((K_CH, COL_CH), jnp.float32),
        pltpu.VMEM((K_CH, COL_CH), jnp.float32),
        pltpu.VMEM((K_CH, COL_CH), jnp.float32),
        pltpu.VMEM((K_CH, COL_CH), jnp.float32),
        pltpu.VMEM((CLS_PER_Q + 8, COL_CH), jnp.float32),
        pltpu.SemaphoreType.DMA,
        pltpu.SemaphoreType.DMA,
        pltpu.SemaphoreType.DMA,
        pltpu.SemaphoreType.DMA,
        pltpu.SemaphoreType.DMA,
        pltpu.SemaphoreType.DMA,
        pltpu.SemaphoreType.DMA,
        pltpu.SemaphoreType.DMA,
    ],
)(_sc_body)


def _update_body(seg_ref, proto_ref, y_ref, ctr_ref, upd_ref, p2_ref):
    ids = _BCU * pl.program_id(0) + lax.broadcasted_iota(
        jnp.int32, (_BCU, 1), 0)
    cnt = jnp.zeros((_BCU, 1), jnp.float32)
    for k in range(N_ROWS // 2048):
        yk = y_ref[0:1, pl.ds(k * 2048, 2048)]
        cnt = cnt + jnp.sum((yk == ids).astype(jnp.float32), axis=1,
                            keepdims=True)
    ctr = ctr_ref[...]
    seg = seg_ref[...]
    proto = proto_ref[...]
    npb = seg / jnp.maximum(cnt, 1.0)
    tot = ctr + cnt
    upd = jnp.where(cnt > 0.0,
                    (ctr * proto + cnt * npb) / jnp.maximum(tot, 1.0),
                    proto)
    upd_ref[...] = upd.astype(jnp.bfloat16)
    p2_ref[...] = jnp.sum(upd * upd, axis=1, keepdims=True)


_BCU = 256


def _update_call(seg, proto, y2d, ctr2):
    return pl.pallas_call(
        _update_body,
        grid=(N_CLS // _BCU,),
        in_specs=[
            pl.BlockSpec((_BCU, EMB), lambda i: (i, 0)),
            pl.BlockSpec((_BCU, EMB), lambda i: (i, 0)),
            pl.BlockSpec((1, N_ROWS), lambda i: (0, 0)),
            pl.BlockSpec((_BCU, 1), lambda i: (i, 0)),
        ],
        out_specs=[
            pl.BlockSpec((_BCU, EMB), lambda i: (i, 0)),
            pl.BlockSpec((_BCU, 1), lambda i: (i, 0)),
        ],
        out_shape=[jax.ShapeDtypeStruct((N_CLS, EMB), jnp.bfloat16),
                   jax.ShapeDtypeStruct((N_CLS, 1), jnp.float32)],
    )(seg, proto, y2d, ctr2)


_BM = 2048
_BC = 512


def _dist_body(x_ref, u_ref, p2_ref, o_ref):
    x = x_ref[...]
    dot = lax.dot_general(x.astype(jnp.bfloat16), u_ref[...],
                          (((1,), (1,)), ((), ())),
                          preferred_element_type=jnp.float32)
    x2 = jnp.sum(x * x, axis=1, keepdims=True)
    sq = x2 + p2_ref[...] - 2.0 * dot
    o_ref[...] = -jnp.maximum(sq, 0.0)


def _dist_call(x, upd, p2r):
    return pl.pallas_call(
        _dist_body,
        grid=(N_ROWS // _BM, N_CLS // _BC),
        in_specs=[
            pl.BlockSpec((_BM, EMB), lambda m, c: (m, 0)),
            pl.BlockSpec((_BC, EMB), lambda m, c: (c, 0)),
            pl.BlockSpec((1, _BC), lambda m, c: (0, c)),
        ],
        out_specs=pl.BlockSpec((_BM, _BC), lambda m, c: (m, c)),
        out_shape=jax.ShapeDtypeStruct((N_ROWS, N_CLS), jnp.float32),
    )(x, upd, p2r)


def kernel(x, y_true, prototypes, counter):
    y2 = y_true.reshape(N_CH, K_CH)
    seg = _sc_segsum(x, y2)
    upd, p2 = _update_call(seg, prototypes, y_true.reshape(1, N_ROWS),
                           counter.reshape(N_CLS, 1))
    return _dist_call(x, upd, p2.reshape(1, N_CLS))


# speedup vs baseline: 1.1151x; 1.1151x over previous
import functools

import jax
import jax.numpy as jnp
from jax import lax
from jax.experimental import pallas as pl
from jax.experimental.pallas import tpu as pltpu
from jax.experimental.pallas import tpu_sc as plsc

N_ROWS = 16384
N_CLS = 2048
EMB = 1024

NUM_SC = 2
NUM_TILES = 16
NUM_WORKERS = NUM_SC * NUM_TILES
LANES = 16

COL_CH = 128
N_CGRP = EMB // COL_CH
N_QGRP = NUM_WORKERS // N_CGRP
CLS_PER_Q = N_CLS // N_QGRP
K_CH = 128
N_CH = N_ROWS // K_CH


def _sc_body(x_hbm, y2_hbm, seg_hbm,
             y_v, buf0, buf1, acc_v, sem0, sem1, ysem0, ysem1):
    c = lax.axis_index("c")
    s = lax.axis_index("s")
    w = c * NUM_TILES + s
    q = w % N_QGRP
    col0 = pl.multiple_of((w // N_QGRP) * COL_CH, COL_CH)
    cls0 = q * CLS_PER_Q
    zero16 = jnp.zeros((LANES,), jnp.float32)
    lane = lax.iota(jnp.int32, LANES)
    cols = [lane + j * LANES for j in range(COL_CH // LANES)]

    def _zacc(i, carry):
        for j in range(COL_CH // LANES):
            acc_v[i, pl.ds(j * LANES, LANES)] = zero16
        return carry
    lax.fori_loop(0, CLS_PER_Q + 8, _zacc, 0)

    NB = 2
    bufs = (buf0, buf1)
    sems = (sem0, sem1)
    ysems = (ysem0, ysem1)

    def _copy(t, p):
        return pltpu.make_async_copy(
            x_hbm.at[pl.ds(t * K_CH, K_CH), pl.ds(col0, COL_CH)],
            bufs[p], sems[p])

    def _ycopy(t, p):
        return pltpu.make_async_copy(y2_hbm.at[t], y_v.at[p], ysems[p])

    def _consume(t, p):
        buf = bufs[p]

        @plsc.parallel_loop(0, K_CH // LANES, unroll=2)
        def _grp(g16):
            yvec = y_v[p, pl.ds(g16 * LANES, LANES)]
            keep = (yvec >= cls0) & (yvec < cls0 + CLS_PER_Q)
            ylvec = jnp.where(keep, yvec - cls0, CLS_PER_Q)
            for l in range(LANES):
                yl = ylvec[l]
                ylv = jnp.full((LANES,), yl, jnp.int32)
                r = g16 * LANES + l
                for j in range(COL_CH // LANES):
                    plsc.addupdate_scatter(
                        acc_v, [ylv, cols[j]],
                        buf[r, pl.ds(j * LANES, LANES)])

    for t0 in range(NB):
        _copy(t0, t0).start()
        _ycopy(t0, t0).start()

    def _step(i, carry):
        for p in range(NB):
            t = NB * i + p
            _copy(t, p).wait()
            _ycopy(t, p).wait()
            _consume(t, p)

            @pl.when(t + NB < N_CH)
            def _(t=t, p=p):
                _copy(t + NB, p).start()
                _ycopy(t + NB, p).start()
        return carry
    lax.fori_loop(0, N_CH // NB, _step, 0)

    pltpu.sync_copy(acc_v.at[pl.ds(0, CLS_PER_Q)],
                    seg_hbm.at[pl.ds(cls0, CLS_PER_Q), pl.ds(col0, COL_CH)])


_sc_segsum = functools.partial(
    pl.kernel,
    out_type=jax.ShapeDtypeStruct((N_CLS, EMB), jnp.float32),
    mesh=plsc.VectorSubcoreMesh(core_axis_name="c", subcore_axis_name="s",
                                num_cores=NUM_SC, num_subcores=NUM_TILES),
    compiler_params=pltpu.CompilerParams(needs_layout_passes=False),
    scratch_types=[
        pltpu.VMEM((2, K_CH), jnp.int32),
        pltpu.VMEM((K_CH, COL_CH), jnp.float32),
        pltpu.VMEM((K_CH, COL_CH), jnp.float32),
        pltpu.VMEM((CLS_PER_Q + 8, COL_CH), jnp.float32),
        pltpu.SemaphoreType.DMA,
        pltpu.SemaphoreType.DMA,
        pltpu.SemaphoreType.DMA,
        pltpu.SemaphoreType.DMA,
    ],
)(_sc_body)


def _update_body(seg_ref, proto_ref, y_ref, ctr_ref, upd_ref, p2_ref):
    ids = _BCU * pl.program_id(0) + lax.broadcasted_iota(
        jnp.int32, (_BCU, 1), 0)
    cnt = jnp.zeros((_BCU, 1), jnp.float32)
    for k in range(N_ROWS // 2048):
        yk = y_ref[0:1, pl.ds(k * 2048, 2048)]
        cnt = cnt + jnp.sum((yk == ids).astype(jnp.float32), axis=1,
                            keepdims=True)
    ctr = ctr_ref[...]
    seg = seg_ref[...]
    proto = proto_ref[...]
    npb = seg / jnp.maximum(cnt, 1.0)
    tot = ctr + cnt
    upd = jnp.where(cnt > 0.0,
                    (ctr * proto + cnt * npb) / jnp.maximum(tot, 1.0),
                    proto)
    upd_ref[...] = upd.astype(jnp.bfloat16)
    p2_ref[...] = jnp.sum(upd * upd, axis=1, keepdims=True)


_BCU = 256


def _update_call(seg, proto, y2d, ctr2):
    return pl.pallas_call(
        _update_body,
        grid=(N_CLS // _BCU,),
        in_specs=[
            pl.BlockSpec((_BCU, EMB), lambda i: (i, 0)),
            pl.BlockSpec((_BCU, EMB), lambda i: (i, 0)),
            pl.BlockSpec((1, N_ROWS), lambda i: (0, 0)),
            pl.BlockSpec((_BCU, 1), lambda i: (i, 0)),
        ],
        out_specs=[
            pl.BlockSpec((_BCU, EMB), lambda i: (i, 0)),
            pl.BlockSpec((_BCU, 1), lambda i: (i, 0)),
        ],
        out_shape=[jax.ShapeDtypeStruct((N_CLS, EMB), jnp.bfloat16),
                   jax.ShapeDtypeStruct((N_CLS, 1), jnp.float32)],
    )(seg, proto, y2d, ctr2)


_BM = 2048
_BC = 512


def _dist_body(x_ref, u_ref, p2_ref, o_ref):
    x = x_ref[...]
    dot = lax.dot_general(x.astype(jnp.bfloat16), u_ref[...],
                          (((1,), (1,)), ((), ())),
                          preferred_element_type=jnp.float32)
    x2 = jnp.sum(x * x, axis=1, keepdims=True)
    sq = x2 + p2_ref[...] - 2.0 * dot
    o_ref[...] = -jnp.maximum(sq, 0.0)


def _dist_call(x, upd, p2r):
    return pl.pallas_call(
        _dist_body,
        grid=(N_ROWS // _BM, N_CLS // _BC),
        in_specs=[
            pl.BlockSpec((_BM, EMB), lambda m, c: (m, 0)),
            pl.BlockSpec((_BC, EMB), lambda m, c: (c, 0)),
            pl.BlockSpec((1, _BC), lambda m, c: (0, c)),
        ],
        out_specs=pl.BlockSpec((_BM, _BC), lambda m, c: (m, c)),
        out_shape=jax.ShapeDtypeStruct((N_ROWS, N_CLS), jnp.float32),
    )(x, upd, p2r)


def kernel(x, y_true, prototypes, counter):
    y2 = y_true.reshape(N_CH, K_CH)
    seg = _sc_segsum(x, y2)
    upd, p2 = _update_call(seg, prototypes, y_true.reshape(1, N_ROWS),
                           counter.reshape(N_CLS, 1))
    return _dist_call(x, upd, p2.reshape(1, N_CLS))
